# baseline (speedup 1.0000x reference)
import dataclasses

import jax
import jax.numpy as jnp
from jax import lax
from jax.experimental import pallas as pl
from jax.experimental.pallas import tpu as pltpu
from jax.experimental.pallas import tpu_sc as plsc

_N = 320000
_R = 10000
_H = 128
_B = 12800
_NB = _N // _B
_W = 128
_RP = 10240

_TILES = 32
_APT = _N // _TILES
_NV = _APT // 16
_G = 128
_SPLIT = 5120
_ACC_R = 5376
_TRASH = 5200
_APS = _ACC_R // 16
_TMAX = 80

_F32 = jnp.float32
_I32 = jnp.int32


def _sc_feat_body(feat_hbm, typ_hbm, seg_hbm, out_hbm,
                  typ_t, seg_t, idxca, seg_lo, seg_hi, rb0, rb1, zbuf, acc,
                  gsem0, gsem1, ssem0, ssem1):
    c = lax.axis_index("c")
    s = lax.axis_index("s")
    base = (c * 16 + s) * _APT

    pltpu.sync_copy(typ_hbm.at[pl.ds(base, _APT)], typ_t)
    pltpu.sync_copy(seg_hbm.at[pl.ds(base, _APT)], seg_t)

    @pl.loop(0, 16)
    def _zb(i):
        @pl.loop(0, _H, step=16)
        def _zl(l):
            zbuf[i, pl.ds(l, 16)] = jnp.zeros((16,), _F32)

    @pl.loop(0, _TMAX * _G, step=16)
    def _pf(i):
        idxca[pl.ds(i, 16)] = jnp.zeros((16,), _I32)

    @pl.loop(0, _TMAX)
    def _pf2(r):
        @pl.loop(0, _G, step=16)
        def _pf2c(l):
            seg_lo[r, pl.ds(l, 16)] = jnp.full((16,), _TRASH, _I32)
            seg_hi[r, pl.ds(l, 16)] = jnp.full((16,), _TRASH, _I32)

    iota16 = lax.broadcasted_iota(_I32, (16,), 0)

    def comp_body(i, carry):
        off, off_lo = carry
        tv = typ_t[pl.ds(i * 16, 16)]
        sv = seg_t[pl.ds(i * 16, 16)]
        m = tv == 1
        mi = m.astype(_I32)
        p = off + plsc.cumsum(mi) - 1
        pr = p // _G
        pc = p - pr * _G
        ivec = iota16 + (base + i * 16)
        in_lo = sv < _SPLIT
        svl = jnp.where(in_lo, sv, _TRASH)
        svh = jnp.where(in_lo, _TRASH, sv - _SPLIT)
        plsc.store_scatter(idxca, [p], ivec, mask=m)
        plsc.store_scatter(seg_lo, [pr, pc], svl, mask=m)
        plsc.store_scatter(seg_hi, [pr, pc], svh, mask=m)
        return off + jnp.sum(mi), off_lo + jnp.sum(mi * in_lo.astype(_I32))

    n, n_lo = lax.fori_loop(0, _NV, comp_body, (0, 0))
    trips = (n + (_G - 1)) // _G
    hi0 = (n_lo + (_G - 1)) // _G
    lo1 = n_lo // _G

    def make_pair(seg_buf):
        def pair(q, carry):
            j0 = 2 * q
            j1 = j0 + 1
            g0 = pltpu.async_copy(
                feat_hbm.at[idxca.at[pl.ds(j0 * _G, _G)]], rb0, gsem0)
            g1 = pltpu.async_copy(
                feat_hbm.at[idxca.at[pl.ds(j1 * _G, _G)]], rb1, gsem1)
            g0.wait()
            s0 = pltpu.async_copy(rb0, acc.at[seg_buf.at[j0]], ssem0,
                                  add=True)
            g1.wait()
            s1 = pltpu.async_copy(rb1, acc.at[seg_buf.at[j1]], ssem1,
                                  add=True)
            s0.wait()
            s1.wait()
            return carry
        return pair

    for p, seg_buf, q_lo, q_hi in ((0, seg_lo, 0, (hi0 + 1) // 2),
                                   (1, seg_hi, lo1 // 2, (trips + 1) // 2)):
        @pl.loop(0, _APS, step=16)
        def _za(k):
            pltpu.sync_copy(zbuf, acc.at[pl.ds(s * _APS + k, 16)])

        plsc.subcore_barrier()
        lax.fori_loop(q_lo, q_hi, make_pair(seg_buf), 0)
        plsc.subcore_barrier()

        pltpu.sync_copy(
            acc.at[pl.ds(s * _APS, _APS)],
            out_hbm.at[pl.ds((c * 2 + p) * _ACC_R + s * _APS, _APS)])


def _sc_feat(node_features, typ, seg):
    mesh = plsc.VectorSubcoreMesh(core_axis_name="c", subcore_axis_name="s")
    cp = pltpu.CompilerParams()
    if "needs_layout_passes" in pltpu.CompilerParams.__dataclass_fields__:
        cp = dataclasses.replace(cp, needs_layout_passes=False)
    return pl.kernel(
        _sc_feat_body,
        out_type=jax.ShapeDtypeStruct((4 * _ACC_R, _H), _F32),
        mesh=mesh,
        scratch_types=[
            pltpu.VMEM((_APT,), _I32),
            pltpu.VMEM((_APT,), _I32),
            pltpu.VMEM((_TMAX * _G,), _I32),
            pltpu.VMEM((_TMAX, _G), _I32),
            pltpu.VMEM((_TMAX, _G), _I32),
            pltpu.VMEM((_G, _H), _F32),
            pltpu.VMEM((_G, _H), _F32),
            pltpu.VMEM((16, _H), _F32),
            pltpu.VMEM_SHARED((_ACC_R, _H), _F32),
            pltpu.SemaphoreType.DMA,
            pltpu.SemaphoreType.DMA,
            pltpu.SemaphoreType.DMA,
            pltpu.SemaphoreType.DMA,
        ],
        compiler_params=cp,
    )(node_features, typ, seg)


def _small_body(seg_ref, tcol_ref, pos16_ref, outS_ref):
    b = pl.program_id(0)

    @pl.when(b == 0)
    def _init():
        outS_ref[...] = jnp.zeros_like(outS_ref)

    seg_row = seg_ref[0]
    tcol = tcol_ref[...]
    pos16 = pos16_ref[...]

    li = lax.broadcasted_iota(_I32, (1, 16), 1)
    sel = jnp.where(li < 3, 1,
          jnp.where(li < 6, 4,
          jnp.where(li == 6, 1,
          jnp.where(li == 8, 0,
          jnp.where(li == 9, 2,
          jnp.where(li == 10, 4, -1))))))
    c7 = (li == 7).astype(_F32)
    t16 = jnp.broadcast_to(tcol, (_B, 16))
    A = pos16 * (t16 == sel).astype(_F32) + c7

    s0 = seg_ref[0, 0, 0]
    sL = seg_ref[0, 0, _B - 1]
    c0 = (s0 // 8) * 8
    nch = (sL - c0) // _W + 1

    def chunk(c, carry):
        cstart = c0 + c * _W
        iw = lax.broadcasted_iota(_I32, (_W, _B), 0)
        ohf = ((seg_row - cstart) == iw).astype(_F32)
        outS_ref[pl.ds(cstart, _W), :] += jnp.dot(
            ohf, A, preferred_element_type=_F32)
        return carry

    lax.fori_loop(0, nch, chunk, 0)


def _comb_body(p_ref, s_ref, out_ref):
    inv = 1.0 / jnp.maximum(s_ref[:, 6:7], 1.0)
    nup = _RP - _SPLIT
    out_ref[0:_SPLIT, :] = ((p_ref[0, 0:_SPLIT, :] + p_ref[2, 0:_SPLIT, :])
                            * inv[0:_SPLIT])
    out_ref[_SPLIT:_RP, :] = ((p_ref[1, 0:nup, :] + p_ref[3, 0:nup, :])
                              * inv[_SPLIT:_RP])


def _epi_body(sT_ref, out_ref):
    def row(i):
        return sT_ref[i:i + 1, :]

    ca_cnt = row(6)
    inv_ca = 1.0 / jnp.maximum(ca_cnt, 1.0)
    cax = row(0) * inv_ca
    cay = row(1) * inv_ca
    caz = row(2) * inv_ca
    inv_cb = 1.0 / jnp.maximum(row(10), 1.0)
    cbx = row(3) * inv_cb
    cby = row(4) * inv_cb
    cbz = row(5) * inv_cb
    no_cb = (jnp.abs(cbx) + jnp.abs(cby) + jnp.abs(cbz)) < 1e-6
    cbx = jnp.where(no_cb, cax, cbx)
    cby = jnp.where(no_cb, cay, cby)
    cbz = jnp.where(no_cb, caz, cbz)

    e1x = cbx - cax
    e1y = cby - cay
    e1z = cbz - caz
    n1 = jnp.sqrt(e1x * e1x + e1y * e1y + e1z * e1z)
    d1 = jnp.maximum(n1, 1e-6)
    ux = e1x / d1
    uy = e1y / d1
    uz = e1z / d1
    n2a = jnp.sqrt(ux * ux + uy * uy)
    use_b = n2a < 1e-6
    e2x = jnp.where(use_b, -uz, uy)
    e2y = jnp.where(use_b, 0.0, -ux)
    e2z = jnp.where(use_b, ux, 0.0)
    n2 = jnp.sqrt(e2x * e2x + e2y * e2y + e2z * e2z)
    d2 = jnp.maximum(n2, 1e-6)
    vx = e2x / d2
    vy = e2y / d2
    vz = e2z / d2
    wx = uy * vz - uz * vy
    wy = uz * vx - ux * vz
    wz = ux * vy - uy * vx

    ridx = lax.broadcasted_iota(_I32, ca_cnt.shape, 1)
    valid = (n1 > 1e-6) & (n2 > 1e-6) & (ridx < _R - 1)

    one = jnp.ones_like(cax)
    zero = jnp.zeros_like(cax)
    built = (ux, vx, wx, uy, vy, wy, uz, vz, wz)
    eye = (one, zero, zero, zero, one, zero, zero, zero, one)
    for k in range(9):
        out_ref[k:k + 1, :] = jnp.where(valid, built[k], eye[k])

    mask = (row(7) >= 3.0) & (row(8) > 0.0) & (ca_cnt > 0.0) & (row(9) > 0.0)
    out_ref[9:10, :] = mask.astype(_F32)
    out_ref[10:11, :] = cax
    out_ref[11:12, :] = cay
    out_ref[12:13, :] = caz
    out_ref[13:14, :] = cbx
    out_ref[14:15, :] = cby
    out_ref[15:16, :] = cbz


def kernel(node_features, node_positions, atom_type_ids, segment_ids):
    seg = segment_ids.astype(_I32)
    typ = atom_type_ids.astype(_I32)
    seg3d = seg.reshape(_NB, 1, _B)
    typ_col = typ.reshape(_N, 1)
    ones10 = jnp.ones((_N, 10), dtype=_F32)
    pos16 = jnp.concatenate([node_positions, node_positions, ones10], axis=1)

    featsum2 = _sc_feat(node_features, typ, seg)

    outS = pl.pallas_call(
        _small_body,
        grid=(_NB,),
        in_specs=[
            pl.BlockSpec((1, 1, _B), lambda b: (b, 0, 0)),
            pl.BlockSpec((_B, 1), lambda b: (b, 0)),
            pl.BlockSpec((_B, 16), lambda b: (b, 0)),
        ],
        out_specs=pl.BlockSpec((_RP, 16), lambda b: (0, 0)),
        out_shape=jax.ShapeDtypeStruct((_RP, 16), _F32),
    )(seg3d, typ_col, pos16)

    outF = pl.pallas_call(
        _comb_body,
        out_shape=jax.ShapeDtypeStruct((_RP, _H), _F32),
    )(featsum2.reshape(4, _ACC_R, _H), outS)

    sT = outS.T
    outT = pl.pallas_call(
        _epi_body,
        out_shape=jax.ShapeDtypeStruct((16, _RP), _F32),
    )(sT)

    residue_features = outF[:_R]
    pos_CA = outT[10:13, :_R].T
    pos_CB = outT[13:16, :_R].T
    frames = outT[0:9, :_R].T.reshape(_R, 3, 3)
    residue_mask = outT[9, :_R] > 0.5
    return (residue_features, pos_CA, pos_CB, frames, segment_ids,
            residue_mask)


# speedup vs baseline: 1.0553x; 1.0553x over previous
import dataclasses

import jax
import jax.numpy as jnp
from jax import lax
from jax.experimental import pallas as pl
from jax.experimental.pallas import tpu as pltpu
from jax.experimental.pallas import tpu_sc as plsc

_N = 320000
_R = 10000
_H = 128
_B = 6400
_NB = _N // _B
_W = 128
_RP = 10240

_TILES = 32
_APT = _N // _TILES
_NV = _APT // 16
_G = 128
_SPLIT = 5120
_ACC_R = 5376
_TRASH = 5200
_APS = _ACC_R // 16
_TMAX = 80

_F32 = jnp.float32
_I32 = jnp.int32


def _sc_feat_body(feat_hbm, typ_hbm, seg_hbm, out_hbm,
                  typ_t, seg_t, idxca, seg_lo, seg_hi, rb0, rb1, zbuf, acc,
                  gsem0, gsem1, ssem0, ssem1):
    c = lax.axis_index("c")
    s = lax.axis_index("s")
    base = (c * 16 + s) * _APT

    pltpu.sync_copy(typ_hbm.at[pl.ds(base, _APT)], typ_t)
    pltpu.sync_copy(seg_hbm.at[pl.ds(base, _APT)], seg_t)

    @pl.loop(0, 16)
    def _zb(i):
        @pl.loop(0, _H, step=16)
        def _zl(l):
            zbuf[i, pl.ds(l, 16)] = jnp.zeros((16,), _F32)

    @pl.loop(0, _TMAX * _G, step=16)
    def _pf(i):
        idxca[pl.ds(i, 16)] = jnp.zeros((16,), _I32)

    @pl.loop(0, _TMAX)
    def _pf2(r):
        @pl.loop(0, _G, step=16)
        def _pf2c(l):
            seg_lo[r, pl.ds(l, 16)] = jnp.full((16,), _TRASH, _I32)
            seg_hi[r, pl.ds(l, 16)] = jnp.full((16,), _TRASH, _I32)

    iota16 = lax.broadcasted_iota(_I32, (16,), 0)

    def comp_body(i, carry):
        off, off_lo = carry
        tv = typ_t[pl.ds(i * 16, 16)]
        sv = seg_t[pl.ds(i * 16, 16)]
        m = tv == 1
        mi = m.astype(_I32)
        p = off + plsc.cumsum(mi) - 1
        pr = p // _G
        pc = p - pr * _G
        ivec = iota16 + (base + i * 16)
        in_lo = sv < _SPLIT
        svl = jnp.where(in_lo, sv, _TRASH)
        svh = jnp.where(in_lo, _TRASH, sv - _SPLIT)
        plsc.store_scatter(idxca, [p], ivec, mask=m)
        plsc.store_scatter(seg_lo, [pr, pc], svl, mask=m)
        plsc.store_scatter(seg_hi, [pr, pc], svh, mask=m)
        return off + jnp.sum(mi), off_lo + jnp.sum(mi * in_lo.astype(_I32))

    n, n_lo = lax.fori_loop(0, _NV, comp_body, (0, 0))
    trips = (n + (_G - 1)) // _G
    hi0 = (n_lo + (_G - 1)) // _G
    lo1 = n_lo // _G

    def make_pair(seg_buf):
        def pair(q, carry):
            j0 = 2 * q
            j1 = j0 + 1
            g0 = pltpu.async_copy(
                feat_hbm.at[idxca.at[pl.ds(j0 * _G, _G)]], rb0, gsem0)
            g1 = pltpu.async_copy(
                feat_hbm.at[idxca.at[pl.ds(j1 * _G, _G)]], rb1, gsem1)
            g0.wait()
            s0 = pltpu.async_copy(rb0, acc.at[seg_buf.at[j0]], ssem0,
                                  add=True)
            g1.wait()
            s1 = pltpu.async_copy(rb1, acc.at[seg_buf.at[j1]], ssem1,
                                  add=True)
            s0.wait()
            s1.wait()
            return carry
        return pair

    for p, seg_buf, q_lo, q_hi in ((0, seg_lo, 0, (hi0 + 1) // 2),
                                   (1, seg_hi, lo1 // 2, (trips + 1) // 2)):
        @pl.loop(0, _APS, step=16)
        def _za(k):
            pltpu.sync_copy(zbuf, acc.at[pl.ds(s * _APS + k, 16)])

        plsc.subcore_barrier()
        lax.fori_loop(q_lo, q_hi, make_pair(seg_buf), 0)
        plsc.subcore_barrier()

        pltpu.sync_copy(
            acc.at[pl.ds(s * _APS, _APS)],
            out_hbm.at[pl.ds((c * 2 + p) * _ACC_R + s * _APS, _APS)])


def _sc_feat(node_features, typ, seg):
    mesh = plsc.VectorSubcoreMesh(core_axis_name="c", subcore_axis_name="s")
    cp = pltpu.CompilerParams()
    if "needs_layout_passes" in pltpu.CompilerParams.__dataclass_fields__:
        cp = dataclasses.replace(cp, needs_layout_passes=False)
    return pl.kernel(
        _sc_feat_body,
        out_type=jax.ShapeDtypeStruct((4 * _ACC_R, _H), _F32),
        mesh=mesh,
        scratch_types=[
            pltpu.VMEM((_APT,), _I32),
            pltpu.VMEM((_APT,), _I32),
            pltpu.VMEM((_TMAX * _G,), _I32),
            pltpu.VMEM((_TMAX, _G), _I32),
            pltpu.VMEM((_TMAX, _G), _I32),
            pltpu.VMEM((_G, _H), _F32),
            pltpu.VMEM((_G, _H), _F32),
            pltpu.VMEM((16, _H), _F32),
            pltpu.VMEM_SHARED((_ACC_R, _H), _F32),
            pltpu.SemaphoreType.DMA,
            pltpu.SemaphoreType.DMA,
            pltpu.SemaphoreType.DMA,
            pltpu.SemaphoreType.DMA,
        ],
        compiler_params=cp,
    )(node_features, typ, seg)


def _small_body(seg_ref, tcol_ref, pos16_ref, outS_ref):
    b = pl.program_id(0)

    @pl.when(b == 0)
    def _init():
        outS_ref[...] = jnp.zeros_like(outS_ref)

    seg_row = seg_ref[0]
    tcol = tcol_ref[...]
    pos16 = pos16_ref[...]

    li = lax.broadcasted_iota(_I32, (1, 16), 1)
    sel = jnp.where(li < 3, 1,
          jnp.where(li < 6, 4,
          jnp.where(li == 6, 1,
          jnp.where(li == 8, 0,
          jnp.where(li == 9, 2,
          jnp.where(li == 10, 4, -1))))))
    c7 = (li == 7).astype(_F32)
    t16 = jnp.broadcast_to(tcol, (_B, 16))
    A = pos16 * (t16 == sel).astype(_F32) + c7

    s0 = seg_ref[0, 0, 0]
    sL = seg_ref[0, 0, _B - 1]
    c0 = (s0 // 8) * 8
    nch = (sL - c0) // _W + 1

    def chunk(c, carry):
        cstart = c0 + c * _W
        iw = lax.broadcasted_iota(_I32, (_W, _B), 0)
        ohf = ((seg_row - cstart) == iw).astype(_F32)
        outS_ref[pl.ds(cstart, _W), :] += jnp.dot(
            ohf, A, preferred_element_type=_F32)
        return carry

    lax.fori_loop(0, nch, chunk, 0)


def _comb_body(p_ref, s_ref, out_ref):
    inv = 1.0 / jnp.maximum(s_ref[:, 6:7], 1.0)
    nup = _RP - _SPLIT
    out_ref[0:_SPLIT, :] = ((p_ref[0, 0:_SPLIT, :] + p_ref[2, 0:_SPLIT, :])
                            * inv[0:_SPLIT])
    out_ref[_SPLIT:_RP, :] = ((p_ref[1, 0:nup, :] + p_ref[3, 0:nup, :])
                              * inv[_SPLIT:_RP])


def _epi_body(sT_ref, out_ref):
    def row(i):
        return sT_ref[i:i + 1, :]

    ca_cnt = row(6)
    inv_ca = 1.0 / jnp.maximum(ca_cnt, 1.0)
    cax = row(0) * inv_ca
    cay = row(1) * inv_ca
    caz = row(2) * inv_ca
    inv_cb = 1.0 / jnp.maximum(row(10), 1.0)
    cbx = row(3) * inv_cb
    cby = row(4) * inv_cb
    cbz = row(5) * inv_cb
    no_cb = (jnp.abs(cbx) + jnp.abs(cby) + jnp.abs(cbz)) < 1e-6
    cbx = jnp.where(no_cb, cax, cbx)
    cby = jnp.where(no_cb, cay, cby)
    cbz = jnp.where(no_cb, caz, cbz)

    e1x = cbx - cax
    e1y = cby - cay
    e1z = cbz - caz
    n1 = jnp.sqrt(e1x * e1x + e1y * e1y + e1z * e1z)
    d1 = jnp.maximum(n1, 1e-6)
    ux = e1x / d1
    uy = e1y / d1
    uz = e1z / d1
    n2a = jnp.sqrt(ux * ux + uy * uy)
    use_b = n2a < 1e-6
    e2x = jnp.where(use_b, -uz, uy)
    e2y = jnp.where(use_b, 0.0, -ux)
    e2z = jnp.where(use_b, ux, 0.0)
    n2 = jnp.sqrt(e2x * e2x + e2y * e2y + e2z * e2z)
    d2 = jnp.maximum(n2, 1e-6)
    vx = e2x / d2
    vy = e2y / d2
    vz = e2z / d2
    wx = uy * vz - uz * vy
    wy = uz * vx - ux * vz
    wz = ux * vy - uy * vx

    ridx = lax.broadcasted_iota(_I32, ca_cnt.shape, 1)
    valid = (n1 > 1e-6) & (n2 > 1e-6) & (ridx < _R - 1)

    one = jnp.ones_like(cax)
    zero = jnp.zeros_like(cax)
    built = (ux, vx, wx, uy, vy, wy, uz, vz, wz)
    eye = (one, zero, zero, zero, one, zero, zero, zero, one)
    for k in range(9):
        out_ref[k:k + 1, :] = jnp.where(valid, built[k], eye[k])

    mask = (row(7) >= 3.0) & (row(8) > 0.0) & (ca_cnt > 0.0) & (row(9) > 0.0)
    out_ref[9:10, :] = mask.astype(_F32)
    out_ref[10:11, :] = cax
    out_ref[11:12, :] = cay
    out_ref[12:13, :] = caz
    out_ref[13:14, :] = cbx
    out_ref[14:15, :] = cby
    out_ref[15:16, :] = cbz


def kernel(node_features, node_positions, atom_type_ids, segment_ids):
    seg = segment_ids.astype(_I32)
    typ = atom_type_ids.astype(_I32)
    seg3d = seg.reshape(_NB, 1, _B)
    typ_col = typ.reshape(_N, 1)
    ones10 = jnp.ones((_N, 10), dtype=_F32)
    pos16 = jnp.concatenate([node_positions, node_positions, ones10], axis=1)

    featsum2 = _sc_feat(node_features, typ, seg)

    outS = pl.pallas_call(
        _small_body,
        grid=(_NB,),
        in_specs=[
            pl.BlockSpec((1, 1, _B), lambda b: (b, 0, 0)),
            pl.BlockSpec((_B, 1), lambda b: (b, 0)),
            pl.BlockSpec((_B, 16), lambda b: (b, 0)),
        ],
        out_specs=pl.BlockSpec((_RP, 16), lambda b: (0, 0)),
        out_shape=jax.ShapeDtypeStruct((_RP, 16), _F32),
    )(seg3d, typ_col, pos16)

    outF = pl.pallas_call(
        _comb_body,
        out_shape=jax.ShapeDtypeStruct((_RP, _H), _F32),
    )(featsum2.reshape(4, _ACC_R, _H), outS)

    sT = outS.T
    outT = pl.pallas_call(
        _epi_body,
        out_shape=jax.ShapeDtypeStruct((16, _RP), _F32),
    )(sT)

    residue_features = outF[:_R]
    pos_CA = outT[10:13, :_R].T
    pos_CB = outT[13:16, :_R].T
    frames = outT[0:9, :_R].T.reshape(_R, 3, 3)
    residue_mask = outT[9, :_R] > 0.5
    return (residue_features, pos_CA, pos_CB, frames, segment_ids,
            residue_mask)
